# baseline (speedup 1.0000x reference)
import functools

import jax
import jax.numpy as jnp
from jax import lax
from jax.experimental import pallas as pl
from jax.experimental.pallas import tpu as pltpu
from jax.experimental.pallas import tpu_sc as plsc

VOCAB = 100000
D = 128
L = 16384

NC = 2
NS = 16
NW = NC * NS
IDS_PER_W = L // NW
CHUNK = 128
NCHUNK = IDS_PER_W // CHUNK
NLANE = 16
NVREG = D // NLANE

BLK = 16384
NBLK_FRONT = 4
SC_NBLK = 2
TAIL_BLK = NBLK_FRONT + SC_NBLK
V_FRONT = NBLK_FRONT * BLK
V_SC = SC_NBLK * BLK
V_TAIL = VOCAB - V_FRONT - V_SC

ROWS_PER_SUB = V_SC // NW
WTILE = 128
NTILES = ROWS_PER_SUB // WTILE
JUNROLL = 16

_sc_mesh = plsc.VectorSubcoreMesh(core_axis_name="c", subcore_axis_name="s")

UNROLL = 4


@functools.partial(
    pl.kernel,
    mesh=_sc_mesh,
    out_type=jax.ShapeDtypeStruct((NW, D), jnp.float32),
    scratch_types=[
        pltpu.VMEM((NCHUNK, CHUNK), jnp.int32),
        pltpu.VMEM((NCHUNK, CHUNK, D), jnp.float32),
        pltpu.VMEM((D,), jnp.float32),
        pltpu.SemaphoreType.DMA,
        pltpu.SemaphoreType.DMA,
        pltpu.SemaphoreType.DMA,
        pltpu.SemaphoreType.DMA,
    ],
)
def _gather_sum(ids_hbm, emb_hbm, out_hbm, idx_v, rows_v, out_v,
                sem0, sem1, sem2, sem3):
    sems = (sem0, sem1, sem2, sem3)
    wid = lax.axis_index("s") * NC + lax.axis_index("c")
    pltpu.sync_copy(ids_hbm.at[wid], idx_v)
    copies = [
        pltpu.async_copy(emb_hbm.at[idx_v.at[k]], rows_v.at[k], sems[k])
        for k in range(NCHUNK)
    ]
    acc = tuple(jnp.zeros((NLANE,), jnp.float32) for _ in range(NVREG))
    for k in range(NCHUNK):
        copies[k].wait()
        buf = rows_v.at[k]

        def body(i, carry):
            for u in range(UNROLL):
                carry = tuple(
                    carry[j] + buf[i * UNROLL + u, pl.ds(j * NLANE, NLANE)]
                    for j in range(NVREG)
                )
            return carry

        acc = lax.fori_loop(0, CHUNK // UNROLL, body, acc)
    for j in range(NVREG):
        out_v[pl.ds(j * NLANE, NLANE)] = acc[j]
    pltpu.sync_copy(out_v, out_hbm.at[wid])


CH = 4096
NCH = VOCAB // CH
V_MAIN = NCH * CH
V_TAIL = VOCAB - V_MAIN
NBUF = 4


def _matvec_body(p_ref, w_ref, bm_ref, bt_ref, om_ref, ot_ref,
                 *scratch):
    bufs = scratch[:NBUF]
    sems = scratch[NBUF:NBUF + NBUF]
    tbuf = scratch[NBUF + NBUF]
    tsem = scratch[NBUF + NBUF + 1]

    tail_copy = pltpu.make_async_copy(
        w_ref.at[pl.ds(V_MAIN, V_TAIL)], tbuf, tsem)
    tail_copy.start()
    copies = []
    for k in range(NBUF):
        c = pltpu.make_async_copy(
            w_ref.at[pl.ds(k * CH, CH)], bufs[k], sems[k])
        c.start()
        copies.append(c)

    s = jnp.sum(p_ref[...], axis=0, keepdims=True)

    for k in range(NCH):
        copies[k].wait()
        mv = lax.dot_general(
            s, bufs[k % NBUF][...], (((1,), (1,)), ((), ())),
            preferred_element_type=jnp.float32,
        )
        om_ref[0:1, pl.ds(k * CH, CH)] = mv + bm_ref[0:1, pl.ds(k * CH, CH)]
        if k + NBUF < NCH:
            c = pltpu.make_async_copy(
                w_ref.at[pl.ds((k + NBUF) * CH, CH)],
                bufs[k % NBUF], sems[k % NBUF])
            c.start()
            copies.append(c)

    tail_copy.wait()
    mvt = lax.dot_general(
        s, tbuf[...], (((1,), (1,)), ((), ())),
        preferred_element_type=jnp.float32,
    )
    ot_ref[...] = mvt + bt_ref[...]


def kernel(context_ids, embedding, W, b):
    ids3 = context_ids.reshape(NW, NCHUNK, CHUNK)
    partials = _gather_sum(ids3, embedding)
    b2 = b.reshape(1, VOCAB)
    out_main, out_tail = pl.pallas_call(
        _matvec_body,
        in_specs=[
            pl.BlockSpec(memory_space=pltpu.MemorySpace.VMEM),
            pl.BlockSpec(memory_space=pl.ANY),
            pl.BlockSpec(memory_space=pltpu.MemorySpace.VMEM),
            pl.BlockSpec(memory_space=pltpu.MemorySpace.VMEM),
        ],
        out_specs=[
            pl.BlockSpec(memory_space=pltpu.MemorySpace.VMEM),
            pl.BlockSpec(memory_space=pltpu.MemorySpace.VMEM),
        ],
        out_shape=[
            jax.ShapeDtypeStruct((1, V_MAIN), jnp.float32),
            jax.ShapeDtypeStruct((1, V_TAIL), jnp.float32),
        ],
        scratch_shapes=(
            [pltpu.VMEM((CH, D), jnp.float32) for _ in range(NBUF)]
            + [pltpu.SemaphoreType.DMA for _ in range(NBUF)]
            + [pltpu.VMEM((V_TAIL, D), jnp.float32), pltpu.SemaphoreType.DMA]
        ),
    )(partials, W, b2[:, :V_MAIN], b2[:, V_MAIN:])
    return jnp.concatenate([out_main, out_tail], axis=1)
